# baseline (speedup 1.0000x reference)
import functools

import jax
import jax.numpy as jnp
from jax import lax
from jax.experimental import pallas as pl
from jax.experimental.pallas import tpu as pltpu
from jax.experimental.pallas import tpu_sc as plsc

_NW = 32
_L = 16


def _sc_body(x_hbm, e_hbm, o_hbm, xv, ov, ev):
    P, C, W = x_hbm.shape
    D = e_hbm.shape[1]
    planes_per_w = P // _NW
    wid = lax.axis_index("s") * 2 + lax.axis_index("c")

    pltpu.sync_copy(e_hbm, ev)

    def plane_body(i, carry):
        p = wid * planes_per_w + i
        pltpu.sync_copy(x_hbm.at[p], xv)

        def wc_body(wc, carry2):
            sl = pl.ds(wc * _L, _L)
            accs = [jnp.zeros((_L,), jnp.float32) for _ in range(D)]
            for c in range(C):
                xvec = xv[c, sl]
                erow = ev[c, :]
                for d in range(D):
                    accs[d] = accs[d] + erow[d] * xvec
            for d in range(D):
                ov[d, sl] = accs[d]
            return carry2

        lax.fori_loop(0, W // _L, wc_body, 0, unroll=False)
        pltpu.sync_copy(ov, o_hbm.at[p])
        return carry

    lax.fori_loop(0, planes_per_w, plane_body, 0, unroll=False)


def kernel(inputs, channel_embeddings):
    B, H, W, C = inputs.shape
    D = channel_embeddings.shape[1]
    P = B * H

    x_t = jnp.transpose(inputs, (0, 1, 3, 2)).reshape(P, C, W)

    sc_call = pl.kernel(
        _sc_body,
        out_type=jax.ShapeDtypeStruct((P, D, W), jnp.float32),
        mesh=plsc.VectorSubcoreMesh(core_axis_name="c", subcore_axis_name="s"),
        scratch_types=[
            pltpu.VMEM((C, W), jnp.float32),
            pltpu.VMEM((D, W), jnp.float32),
            pltpu.VMEM((C, D), jnp.float32),
        ],
        compiler_params=pltpu.CompilerParams(use_tc_tiling_on_sc=True),
    )
    out_t = sc_call(x_t, channel_embeddings)
    return jnp.transpose(out_t.reshape(B, H, D, W), (0, 1, 3, 2))


# speedup vs baseline: 30.8456x; 30.8456x over previous
import jax
import jax.numpy as jnp
from jax.experimental import pallas as pl
from jax.experimental.pallas import tpu as pltpu

_BLOCK_H = 112


def _contract_kernel(xa_ref, xb_ref, ea_ref, eb_ref, o_ref):
    ea = ea_ref[...]
    eb = eb_ref[...]
    for h in range(xa_ref.shape[1]):
        xa = xa_ref[0, h].astype(jnp.bfloat16)
        xb = xb_ref[0, h].astype(jnp.bfloat16)
        dims = (((1,), (0,)), ((), ()))
        o_ref[0, h] = jax.lax.dot_general(
            ea, xa, dims, preferred_element_type=jnp.float32
        ) + jax.lax.dot_general(
            eb, xb, dims, preferred_element_type=jnp.float32
        )


def kernel(inputs, channel_embeddings):
    B, H, W, C = inputs.shape
    D = channel_embeddings.shape[1]
    Ch = C // 2

    x_t = jnp.transpose(inputs, (0, 1, 3, 2))
    e_t = jnp.transpose(channel_embeddings, (1, 0)).astype(jnp.bfloat16)
    e_a = e_t[:, :Ch]
    e_b = e_t[:, Ch:]

    out_t = pl.pallas_call(
        _contract_kernel,
        grid=(B, H // _BLOCK_H),
        in_specs=[
            pl.BlockSpec((1, _BLOCK_H, Ch, W), lambda b, h: (b, h, 0, 0)),
            pl.BlockSpec((1, _BLOCK_H, Ch, W), lambda b, h: (b, h, 1, 0)),
            pl.BlockSpec((D, Ch), lambda b, h: (0, 0)),
            pl.BlockSpec((D, Ch), lambda b, h: (0, 0)),
        ],
        out_specs=pl.BlockSpec((1, _BLOCK_H, D, W), lambda b, h: (b, h, 0, 0)),
        out_shape=jax.ShapeDtypeStruct((B, H, D, W), jnp.float32),
        compiler_params=pltpu.CompilerParams(
            dimension_semantics=("arbitrary", "arbitrary"),
        ),
    )(x_t, x_t, e_a, e_b)
    return jnp.transpose(out_t, (0, 1, 3, 2))
